# baseline (speedup 1.0000x reference)
import jax
import jax.numpy as jnp
from jax.experimental import pallas as pl
from jax.experimental.pallas import tpu as pltpu

_N_SPLIT = 8


def _copy_body(in_ref, out_ref, sems):
    n_rows = in_ref.shape[0]
    chunk = n_rows // _N_SPLIT
    copies = []
    for i in range(_N_SPLIT):
        lo = i * chunk
        hi = n_rows if i == _N_SPLIT - 1 else lo + chunk
        copies.append(
            pltpu.make_async_copy(
                in_ref.at[pl.ds(lo, hi - lo), :],
                out_ref.at[pl.ds(lo, hi - lo), :],
                sems.at[i],
            )
        )
    for c in copies:
        c.start()
    for c in copies:
        c.wait()


def kernel(g, inp_h):
    return pl.pallas_call(
        _copy_body,
        out_shape=jax.ShapeDtypeStruct(inp_h.shape, inp_h.dtype),
        in_specs=[pl.BlockSpec(memory_space=pl.ANY)],
        out_specs=pl.BlockSpec(memory_space=pl.ANY),
        scratch_shapes=[pltpu.SemaphoreType.DMA((_N_SPLIT,))],
    )(inp_h)


# speedup vs baseline: 47.0502x; 47.0502x over previous
import jax
import jax.numpy as jnp
from jax.experimental import pallas as pl
from jax.experimental.pallas import tpu as pltpu

_BLOCK_ROWS = 10000


def _copy_body(in_ref, out_ref):
    out_ref[...] = in_ref[...]


def kernel(g, inp_h):
    n_rows, n_cols = inp_h.shape
    grid = n_rows // _BLOCK_ROWS
    return pl.pallas_call(
        _copy_body,
        out_shape=jax.ShapeDtypeStruct(inp_h.shape, inp_h.dtype),
        grid=(grid,),
        in_specs=[pl.BlockSpec((_BLOCK_ROWS, n_cols), lambda i: (i, 0))],
        out_specs=pl.BlockSpec((_BLOCK_ROWS, n_cols), lambda i: (i, 0)),
    )(inp_h)
